# baseline (speedup 1.0000x reference)
import dataclasses
import functools

import jax
import jax.numpy as jnp
from jax import lax
from jax.experimental import pallas as pl
from jax.experimental.pallas import tpu as pltpu
from jax.experimental.pallas import tpu_sc as plsc

_N = 4096
_H = 512
_T = 16384
_BLK = 256

_NW = 32
_PPW = _T // _NW
_CH = 32
_NCH = _PPW // _CH
_L = 16


def _mm_body(a_ref, b_ref, o_ref):
    o_ref[...] = jnp.dot(a_ref[...], b_ref[...],
                         preferred_element_type=jnp.float32)


def _matmul(a, b):
    n, d = a.shape
    h = b.shape[1]
    return pl.pallas_call(
        _mm_body,
        grid=(1,),
        in_specs=[
            pl.BlockSpec((n, d), lambda i: (0, 0)),
            pl.BlockSpec((d, h), lambda i: (0, 0)),
        ],
        out_specs=pl.BlockSpec((n, h), lambda i: (0, 0)),
        out_shape=jax.ShapeDtypeStruct((n, h), jnp.float32),
    )(a, b)


def _gcn_body(xw_ref, sim_ref, o_ref, deg_ref):
    s = sim_ref[...]
    acc = jnp.dot(s, xw_ref[...], preferred_element_type=jnp.float32)
    deg = jnp.sum(s, axis=1, keepdims=True) + 1e-8
    deg_ref[...] = deg
    o_ref[...] = jnp.maximum(acc / deg, 0.0)


def _gcn_body_deg(xw_ref, sim_ref, deg_ref, o_ref):
    acc = jnp.dot(sim_ref[...], xw_ref[...],
                  preferred_element_type=jnp.float32)
    o_ref[...] = jnp.maximum(acc / deg_ref[...], 0.0)


def _gcn(sim, x, w, deg=None):
    n, d = x.shape
    h = w.shape[1]
    xw = _matmul(x, w)
    if deg is None:
        out, deg_out = pl.pallas_call(
            _gcn_body,
            grid=(n // _BLK,),
            in_specs=[
                pl.BlockSpec((n, h), lambda i: (0, 0)),
                pl.BlockSpec((_BLK, n), lambda i: (i, 0)),
            ],
            out_specs=[
                pl.BlockSpec((_BLK, h), lambda i: (i, 0)),
                pl.BlockSpec((_BLK, 1), lambda i: (i, 0)),
            ],
            out_shape=[
                jax.ShapeDtypeStruct((n, h), jnp.float32),
                jax.ShapeDtypeStruct((n, 1), jnp.float32),
            ],
            compiler_params=pltpu.CompilerParams(
                dimension_semantics=("arbitrary",)),
        )(xw, sim)
        return out, deg_out
    out = pl.pallas_call(
        _gcn_body_deg,
        grid=(n // _BLK,),
        in_specs=[
            pl.BlockSpec((n, h), lambda i: (0, 0)),
            pl.BlockSpec((_BLK, n), lambda i: (i, 0)),
            pl.BlockSpec((_BLK, 1), lambda i: (i, 0)),
        ],
        out_specs=pl.BlockSpec((_BLK, h), lambda i: (i, 0)),
        out_shape=jax.ShapeDtypeStruct((n, h), jnp.float32),
        compiler_params=pltpu.CompilerParams(
            dimension_semantics=("arbitrary",)),
    )(xw, sim, deg)
    return out


def _fuse_body(e1_ref, e2_ref, wa_ref, va_ref, wf_ref, w1a_ref, w1b_ref,
               p_ref, l_ref):
    e1 = e1_ref[...]
    e2 = e2_ref[...]
    wa = wa_ref[...]
    t1 = jnp.tanh(jnp.dot(e1, wa, preferred_element_type=jnp.float32))
    t2 = jnp.tanh(jnp.dot(e2, wa, preferred_element_type=jnp.float32))
    s1 = jnp.dot(t1, va_ref[...], preferred_element_type=jnp.float32)
    s2 = jnp.dot(t2, va_ref[...], preferred_element_type=jnp.float32)
    m = jnp.maximum(s1, s2)
    a1 = jnp.exp(s1 - m)
    a2 = jnp.exp(s2 - m)
    den = a1 + a2
    c1 = (a1 / den) * e1 + (a2 / den) * e2
    c2 = jnp.maximum(jnp.dot(c1, wf_ref[...],
                             preferred_element_type=jnp.float32), 0.0)
    p_ref[...] = (jnp.dot(c1, w1a_ref[...], preferred_element_type=jnp.float32)
                  + jnp.dot(c2, w1b_ref[...],
                            preferred_element_type=jnp.float32))

    q1 = jnp.sum(e1 * e1, axis=1, keepdims=True)
    q2 = jnp.sum(e2 * e2, axis=1, keepdims=True)
    dq = jnp.sum(e1 * e2, axis=1, keepdims=True)
    cos = dq / ((jnp.sqrt(q1) + 1e-8) * (jnp.sqrt(q2) + 1e-8))
    ls = jnp.minimum(cos, 0.0) - jnp.log(1.0 + jnp.exp(-jnp.abs(cos)))
    part = -jnp.sum(ls, axis=0, keepdims=True) / _N

    @pl.when(pl.program_id(0) == 0)
    def _():
        l_ref[...] = jnp.zeros_like(l_ref)

    l_ref[...] += part


def _fuse_proj(e1, e2, wa, va, wf, w1a, w1b):
    n, h = e1.shape
    return pl.pallas_call(
        _fuse_body,
        grid=(n // _BLK,),
        in_specs=[
            pl.BlockSpec((_BLK, h), lambda i: (i, 0)),
            pl.BlockSpec((_BLK, h), lambda i: (i, 0)),
            pl.BlockSpec((h, h), lambda i: (0, 0)),
            pl.BlockSpec((h, 1), lambda i: (0, 0)),
            pl.BlockSpec((h, h), lambda i: (0, 0)),
            pl.BlockSpec((h, h), lambda i: (0, 0)),
            pl.BlockSpec((h, h), lambda i: (0, 0)),
        ],
        out_specs=[
            pl.BlockSpec((_BLK, h), lambda i: (i, 0)),
            pl.BlockSpec((1, 1), lambda i: (0, 0)),
        ],
        out_shape=[
            jax.ShapeDtypeStruct((n, h), jnp.float32),
            jax.ShapeDtypeStruct((1, 1), jnp.float32),
        ],
        compiler_params=pltpu.CompilerParams(
            dimension_semantics=("arbitrary",)),
    )(e1, e2, wa, va, wf, w1a, w1b)


def _pair_predict(pm, pd, mi, di, b1, w2, b2v):
    mesh = plsc.VectorSubcoreMesh(core_axis_name="c", subcore_axis_name="s")
    cp = pltpu.CompilerParams()
    if "needs_layout_passes" in pltpu.CompilerParams.__dataclass_fields__:
        cp = dataclasses.replace(cp, needs_layout_passes=False)

    @pl.kernel(
        compiler_params=cp,
        out_type=jax.ShapeDtypeStruct((_T,), jnp.float32),
        mesh=mesh,
        scratch_types=[
            pltpu.VMEM((_PPW,), jnp.int32),
            pltpu.VMEM((_PPW,), jnp.int32),
            pltpu.VMEM((2, _CH, _H), jnp.float32),
            pltpu.VMEM((2, _CH, _H), jnp.float32),
            pltpu.VMEM((_PPW,), jnp.float32),
            pltpu.VMEM((_H,), jnp.float32),
            pltpu.VMEM((_H,), jnp.float32),
            pltpu.VMEM((_L,), jnp.float32),
            pltpu.SemaphoreType.DMA,
            pltpu.SemaphoreType.DMA,
            pltpu.SemaphoreType.DMA,
            pltpu.SemaphoreType.DMA,
        ],
    )
    def body(pm_hbm, pd_hbm, mi_hbm, di_hbm, b1_hbm, w2_hbm, b2_hbm, o_hbm,
             mi_v, di_v, pm_v, pd_v, out_v, b1_v, w2_v, b2_v,
             sm0, sm1, sd0, sd1):
        wid = lax.axis_index("s") * 2 + lax.axis_index("c")
        base = wid * _PPW
        pltpu.sync_copy(mi_hbm.at[pl.ds(base, _PPW)], mi_v)
        pltpu.sync_copy(di_hbm.at[pl.ds(base, _PPW)], di_v)
        pltpu.sync_copy(b1_hbm, b1_v)
        pltpu.sync_copy(w2_hbm, w2_v)
        pltpu.sync_copy(b2_hbm, b2_v)
        lane = lax.iota(jnp.int32, _L)
        b2vec = b2_v[...]
        sems = ((sm0, sd0), (sm1, sd1))
        handles = [None, None]

        def start(c, b):
            sl = pl.ds(c * _CH, _CH)
            hm = pltpu.async_copy(pm_hbm.at[mi_v.at[sl]], pm_v.at[b],
                                  sems[b][0])
            hd = pltpu.async_copy(pd_hbm.at[di_v.at[sl]], pd_v.at[b],
                                  sems[b][1])
            handles[b] = (hm, hd)

        start(0, 0)
        for c in range(_NCH):
            b = c & 1
            handles[b][0].wait()
            handles[b][1].wait()
            if c + 1 < _NCH:
                start(c + 1, 1 - b)

            @pl.loop(0, _CH, step=_L)
            def _(g0, _c=c, _b=b):
                def pair(p, sv):
                    row = g0 + p
                    acc = jnp.zeros((_L,), jnp.float32)
                    for j in range(_H // _L):
                        pmj = pm_v[_b, row, pl.ds(j * _L, _L)]
                        pdj = pd_v[_b, row, pl.ds(j * _L, _L)]
                        hj = jnp.maximum(
                            pmj + pdj + b1_v[pl.ds(j * _L, _L)], 0.0)
                        acc = acc + hj * w2_v[pl.ds(j * _L, _L)]
                    s = jnp.sum(acc)
                    return jnp.where(lane == p, s, sv)

                sv = lax.fori_loop(0, _L, pair,
                                   jnp.zeros((_L,), jnp.float32))
                logit = sv + b2vec
                out_v[pl.ds(_c * _CH + g0, _L)] = 1.0 / (1.0 + jnp.exp(-logit))

        pltpu.sync_copy(out_v, o_hbm.at[pl.ds(base, _PPW)])

    return body(pm, pd, mi, di, b1, w2, b2v)


def kernel(mm_sim, dd_sim, xm, xd, train_data, Wm1, Wm2, Wd1, Wd2,
           Wa_m, va_m, Wa_d, va_d, Wf_m, Wf_d, W1, b1, W2, b2):
    m_idx = train_data[:, 0].astype(jnp.int32)
    d_idx = train_data[:, 1].astype(jnp.int32)

    em1, deg_m = _gcn(mm_sim, xm, Wm1)
    em2 = _gcn(mm_sim, em1, Wm2, deg_m)
    ed1, deg_d = _gcn(dd_sim, xd, Wd1)
    ed2 = _gcn(dd_sim, ed1, Wd2, deg_d)

    pm, lossc = _fuse_proj(em1, em2, Wa_m, va_m.reshape(_H, 1), Wf_m,
                           W1[0:_H], W1[_H:2 * _H])
    pd, lossd = _fuse_proj(ed1, ed2, Wa_d, va_d.reshape(_H, 1), Wf_d,
                           W1[2 * _H:3 * _H], W1[3 * _H:4 * _H])

    pre = _pair_predict(pm, pd, m_idx, d_idx, b1, W2[:, 0],
                        jnp.broadcast_to(b2, (_L,)))
    return (pre, lossc[0, 0], lossd[0, 0])


# speedup vs baseline: 1.1397x; 1.1397x over previous
import dataclasses
import functools

import jax
import jax.numpy as jnp
from jax import lax
from jax.experimental import pallas as pl
from jax.experimental.pallas import tpu as pltpu
from jax.experimental.pallas import tpu_sc as plsc

_N = 4096
_H = 512
_T = 16384
_BLK = 256

_NW = 32
_PPW = _T // _NW
_CH = 32
_NCH = _PPW // _CH
_L = 16


_GBLK = 512


def _gcn1_body(x_ref, w_ref, sim_ref, o_ref, deg_ref, xw_ref):
    @pl.when(pl.program_id(0) == 0)
    def _():
        xw_ref[...] = jnp.dot(x_ref[...], w_ref[...],
                              preferred_element_type=jnp.float32)

    s = sim_ref[...]
    acc = jnp.dot(s, xw_ref[...], preferred_element_type=jnp.float32)
    deg = jnp.sum(s, axis=1, keepdims=True) + 1e-8
    deg_ref[...] = deg
    o_ref[...] = jnp.maximum(acc / deg, 0.0)


def _gcn2_body(x_ref, w_ref, sim_ref, deg_ref, o_ref, xw_ref):
    @pl.when(pl.program_id(0) == 0)
    def _():
        xw_ref[...] = jnp.dot(x_ref[...], w_ref[...],
                              preferred_element_type=jnp.float32)

    acc = jnp.dot(sim_ref[...], xw_ref[...],
                  preferred_element_type=jnp.float32)
    o_ref[...] = jnp.maximum(acc / deg_ref[...], 0.0)


def _gcn(sim, x, w, deg=None):
    n, d = x.shape
    h = w.shape[1]
    full_x = pl.BlockSpec((n, d), lambda i: (0, 0))
    full_w = pl.BlockSpec((d, h), lambda i: (0, 0))
    sim_spec = pl.BlockSpec((_GBLK, n), lambda i: (i, 0))
    row_spec = pl.BlockSpec((_GBLK, h), lambda i: (i, 0))
    deg_spec = pl.BlockSpec((_GBLK, 1), lambda i: (i, 0))
    params = pltpu.CompilerParams(dimension_semantics=("arbitrary",))
    scratch = [pltpu.VMEM((n, h), jnp.float32)]
    if deg is None:
        return pl.pallas_call(
            _gcn1_body,
            grid=(n // _GBLK,),
            in_specs=[full_x, full_w, sim_spec],
            out_specs=[row_spec, deg_spec],
            out_shape=[
                jax.ShapeDtypeStruct((n, h), jnp.float32),
                jax.ShapeDtypeStruct((n, 1), jnp.float32),
            ],
            scratch_shapes=scratch,
            compiler_params=params,
        )(x, w, sim)
    return pl.pallas_call(
        _gcn2_body,
        grid=(n // _GBLK,),
        in_specs=[full_x, full_w, sim_spec, deg_spec],
        out_specs=row_spec,
        out_shape=jax.ShapeDtypeStruct((n, h), jnp.float32),
        scratch_shapes=scratch,
        compiler_params=params,
    )(x, w, sim, deg)


def _fuse_body(e1_ref, e2_ref, wa_ref, va_ref, wf_ref, w1a_ref, w1b_ref,
               p_ref, l_ref):
    e1 = e1_ref[...]
    e2 = e2_ref[...]
    wa = wa_ref[...]
    t1 = jnp.tanh(jnp.dot(e1, wa, preferred_element_type=jnp.float32))
    t2 = jnp.tanh(jnp.dot(e2, wa, preferred_element_type=jnp.float32))
    s1 = jnp.dot(t1, va_ref[...], preferred_element_type=jnp.float32)
    s2 = jnp.dot(t2, va_ref[...], preferred_element_type=jnp.float32)
    m = jnp.maximum(s1, s2)
    a1 = jnp.exp(s1 - m)
    a2 = jnp.exp(s2 - m)
    den = a1 + a2
    c1 = (a1 / den) * e1 + (a2 / den) * e2
    c2 = jnp.maximum(jnp.dot(c1, wf_ref[...],
                             preferred_element_type=jnp.float32), 0.0)
    p_ref[...] = (jnp.dot(c1, w1a_ref[...], preferred_element_type=jnp.float32)
                  + jnp.dot(c2, w1b_ref[...],
                            preferred_element_type=jnp.float32))

    q1 = jnp.sum(e1 * e1, axis=1, keepdims=True)
    q2 = jnp.sum(e2 * e2, axis=1, keepdims=True)
    dq = jnp.sum(e1 * e2, axis=1, keepdims=True)
    cos = dq / ((jnp.sqrt(q1) + 1e-8) * (jnp.sqrt(q2) + 1e-8))
    ls = jnp.minimum(cos, 0.0) - jnp.log(1.0 + jnp.exp(-jnp.abs(cos)))
    part = -jnp.sum(ls, axis=0, keepdims=True) / _N

    @pl.when(pl.program_id(0) == 0)
    def _():
        l_ref[...] = jnp.zeros_like(l_ref)

    l_ref[...] += part


def _fuse_proj(e1, e2, wa, va, wf, w1a, w1b):
    n, h = e1.shape
    return pl.pallas_call(
        _fuse_body,
        grid=(n // _BLK,),
        in_specs=[
            pl.BlockSpec((_BLK, h), lambda i: (i, 0)),
            pl.BlockSpec((_BLK, h), lambda i: (i, 0)),
            pl.BlockSpec((h, h), lambda i: (0, 0)),
            pl.BlockSpec((h, 1), lambda i: (0, 0)),
            pl.BlockSpec((h, h), lambda i: (0, 0)),
            pl.BlockSpec((h, h), lambda i: (0, 0)),
            pl.BlockSpec((h, h), lambda i: (0, 0)),
        ],
        out_specs=[
            pl.BlockSpec((_BLK, h), lambda i: (i, 0)),
            pl.BlockSpec((1, 1), lambda i: (0, 0)),
        ],
        out_shape=[
            jax.ShapeDtypeStruct((n, h), jnp.float32),
            jax.ShapeDtypeStruct((1, 1), jnp.float32),
        ],
        compiler_params=pltpu.CompilerParams(
            dimension_semantics=("arbitrary",)),
    )(e1, e2, wa, va, wf, w1a, w1b)


def _pair_predict(pm, pd, mi, di, b1, w2, b2v):
    mesh = plsc.VectorSubcoreMesh(core_axis_name="c", subcore_axis_name="s")
    cp = pltpu.CompilerParams()
    if "needs_layout_passes" in pltpu.CompilerParams.__dataclass_fields__:
        cp = dataclasses.replace(cp, needs_layout_passes=False)

    @pl.kernel(
        compiler_params=cp,
        out_type=jax.ShapeDtypeStruct((_T,), jnp.float32),
        mesh=mesh,
        scratch_types=[
            pltpu.VMEM((_PPW,), jnp.int32),
            pltpu.VMEM((_PPW,), jnp.int32),
            pltpu.VMEM((2, _CH, _H), jnp.float32),
            pltpu.VMEM((2, _CH, _H), jnp.float32),
            pltpu.VMEM((_PPW,), jnp.float32),
            pltpu.VMEM((_H,), jnp.float32),
            pltpu.VMEM((_H,), jnp.float32),
            pltpu.VMEM((_L,), jnp.float32),
            pltpu.SemaphoreType.DMA,
            pltpu.SemaphoreType.DMA,
            pltpu.SemaphoreType.DMA,
            pltpu.SemaphoreType.DMA,
        ],
    )
    def body(pm_hbm, pd_hbm, mi_hbm, di_hbm, b1_hbm, w2_hbm, b2_hbm, o_hbm,
             mi_v, di_v, pm_v, pd_v, out_v, b1_v, w2_v, b2_v,
             sm0, sm1, sd0, sd1):
        wid = lax.axis_index("s") * 2 + lax.axis_index("c")
        base = wid * _PPW
        pltpu.sync_copy(mi_hbm.at[pl.ds(base, _PPW)], mi_v)
        pltpu.sync_copy(di_hbm.at[pl.ds(base, _PPW)], di_v)
        pltpu.sync_copy(b1_hbm, b1_v)
        pltpu.sync_copy(w2_hbm, w2_v)
        pltpu.sync_copy(b2_hbm, b2_v)
        lane = lax.iota(jnp.int32, _L)
        b2vec = b2_v[...]
        sems = ((sm0, sd0), (sm1, sd1))
        handles = [None, None]

        def start(c, b):
            sl = pl.ds(c * _CH, _CH)
            hm = pltpu.async_copy(pm_hbm.at[mi_v.at[sl]], pm_v.at[b],
                                  sems[b][0])
            hd = pltpu.async_copy(pd_hbm.at[di_v.at[sl]], pd_v.at[b],
                                  sems[b][1])
            handles[b] = (hm, hd)

        start(0, 0)
        for c in range(_NCH):
            b = c & 1
            handles[b][0].wait()
            handles[b][1].wait()
            if c + 1 < _NCH:
                start(c + 1, 1 - b)

            @pl.loop(0, _CH, step=_L)
            def _(g0, _c=c, _b=b):
                def pair(p, sv):
                    row = g0 + p
                    acc = jnp.zeros((_L,), jnp.float32)
                    for j in range(_H // _L):
                        pmj = pm_v[_b, row, pl.ds(j * _L, _L)]
                        pdj = pd_v[_b, row, pl.ds(j * _L, _L)]
                        hj = jnp.maximum(
                            pmj + pdj + b1_v[pl.ds(j * _L, _L)], 0.0)
                        acc = acc + hj * w2_v[pl.ds(j * _L, _L)]
                    s = jnp.sum(acc)
                    return jnp.where(lane == p, s, sv)

                sv = lax.fori_loop(0, _L, pair,
                                   jnp.zeros((_L,), jnp.float32))
                logit = sv + b2vec
                out_v[pl.ds(_c * _CH + g0, _L)] = 1.0 / (1.0 + jnp.exp(-logit))

        pltpu.sync_copy(out_v, o_hbm.at[pl.ds(base, _PPW)])

    return body(pm, pd, mi, di, b1, w2, b2v)


def kernel(mm_sim, dd_sim, xm, xd, train_data, Wm1, Wm2, Wd1, Wd2,
           Wa_m, va_m, Wa_d, va_d, Wf_m, Wf_d, W1, b1, W2, b2):
    m_idx = train_data[:, 0].astype(jnp.int32)
    d_idx = train_data[:, 1].astype(jnp.int32)

    em1, deg_m = _gcn(mm_sim, xm, Wm1)
    em2 = _gcn(mm_sim, em1, Wm2, deg=deg_m)
    ed1, deg_d = _gcn(dd_sim, xd, Wd1)
    ed2 = _gcn(dd_sim, ed1, Wd2, deg=deg_d)

    pm, lossc = _fuse_proj(em1, em2, Wa_m, va_m.reshape(_H, 1), Wf_m,
                           W1[0:_H], W1[_H:2 * _H])
    pd, lossd = _fuse_proj(ed1, ed2, Wa_d, va_d.reshape(_H, 1), Wf_d,
                           W1[2 * _H:3 * _H], W1[3 * _H:4 * _H])

    pre = _pair_predict(pm, pd, m_idx, d_idx, b1, W2[:, 0],
                        jnp.broadcast_to(b2, (_L,)))
    return (pre, lossc[0, 0], lossd[0, 0])


# speedup vs baseline: 1.1749x; 1.0309x over previous
import dataclasses
import functools

import jax
import jax.numpy as jnp
from jax import lax
from jax.experimental import pallas as pl
from jax.experimental.pallas import tpu as pltpu
from jax.experimental.pallas import tpu_sc as plsc

_N = 4096
_H = 512
_T = 16384
_BLK = 256

_NW = 32
_PPW = _T // _NW
_CH = 32
_NCH = _PPW // _CH
_L = 16


_GBLK = 512


def _gcn1_body(x_ref, w_ref, sim_ref, o_ref, deg_ref, xw_ref):
    @pl.when(pl.program_id(0) == 0)
    def _():
        xw_ref[...] = jnp.dot(x_ref[...], w_ref[...],
                              preferred_element_type=jnp.float32)

    s = sim_ref[...]
    acc = jnp.dot(s, xw_ref[...], preferred_element_type=jnp.float32)
    deg = jnp.sum(s, axis=1, keepdims=True) + 1e-8
    deg_ref[...] = deg
    o_ref[...] = jnp.maximum(acc / deg, 0.0)


def _gcn2_body(x_ref, w_ref, sim_ref, deg_ref, o_ref, xw_ref):
    @pl.when(pl.program_id(0) == 0)
    def _():
        xw_ref[...] = jnp.dot(x_ref[...], w_ref[...],
                              preferred_element_type=jnp.float32)

    acc = jnp.dot(sim_ref[...], xw_ref[...],
                  preferred_element_type=jnp.float32)
    o_ref[...] = jnp.maximum(acc / deg_ref[...], 0.0)


def _gcn(sim, x, w, deg=None):
    n, d = x.shape
    h = w.shape[1]
    full_x = pl.BlockSpec((n, d), lambda i: (0, 0))
    full_w = pl.BlockSpec((d, h), lambda i: (0, 0))
    sim_spec = pl.BlockSpec((_GBLK, n), lambda i: (i, 0))
    row_spec = pl.BlockSpec((_GBLK, h), lambda i: (i, 0))
    deg_spec = pl.BlockSpec((_GBLK, 1), lambda i: (i, 0))
    params = pltpu.CompilerParams(dimension_semantics=("arbitrary",))
    scratch = [pltpu.VMEM((n, h), jnp.float32)]
    if deg is None:
        return pl.pallas_call(
            _gcn1_body,
            grid=(n // _GBLK,),
            in_specs=[full_x, full_w, sim_spec],
            out_specs=[row_spec, deg_spec],
            out_shape=[
                jax.ShapeDtypeStruct((n, h), jnp.float32),
                jax.ShapeDtypeStruct((n, 1), jnp.float32),
            ],
            scratch_shapes=scratch,
            compiler_params=params,
        )(x, w, sim)
    return pl.pallas_call(
        _gcn2_body,
        grid=(n // _GBLK,),
        in_specs=[full_x, full_w, sim_spec, deg_spec],
        out_specs=row_spec,
        out_shape=jax.ShapeDtypeStruct((n, h), jnp.float32),
        scratch_shapes=scratch,
        compiler_params=params,
    )(x, w, sim, deg)


def _fuse_body(e1_ref, e2_ref, wa_ref, va_ref, wf_ref, w1a_ref, w1b_ref,
               bias_ref, p_ref, l_ref):
    e1 = e1_ref[...]
    e2 = e2_ref[...]
    wa = wa_ref[...]
    t1 = jnp.tanh(jnp.dot(e1, wa, preferred_element_type=jnp.float32))
    t2 = jnp.tanh(jnp.dot(e2, wa, preferred_element_type=jnp.float32))
    s1 = jnp.dot(t1, va_ref[...], preferred_element_type=jnp.float32)
    s2 = jnp.dot(t2, va_ref[...], preferred_element_type=jnp.float32)
    m = jnp.maximum(s1, s2)
    a1 = jnp.exp(s1 - m)
    a2 = jnp.exp(s2 - m)
    den = a1 + a2
    c1 = (a1 / den) * e1 + (a2 / den) * e2
    c2 = jnp.maximum(jnp.dot(c1, wf_ref[...],
                             preferred_element_type=jnp.float32), 0.0)
    p_ref[...] = (jnp.dot(c1, w1a_ref[...], preferred_element_type=jnp.float32)
                  + jnp.dot(c2, w1b_ref[...],
                            preferred_element_type=jnp.float32)
                  + bias_ref[...])

    q1 = jnp.sum(e1 * e1, axis=1, keepdims=True)
    q2 = jnp.sum(e2 * e2, axis=1, keepdims=True)
    dq = jnp.sum(e1 * e2, axis=1, keepdims=True)
    cos = dq / ((jnp.sqrt(q1) + 1e-8) * (jnp.sqrt(q2) + 1e-8))
    ls = jnp.minimum(cos, 0.0) - jnp.log(1.0 + jnp.exp(-jnp.abs(cos)))
    part = -jnp.sum(ls, axis=0, keepdims=True) / _N

    @pl.when(pl.program_id(0) == 0)
    def _():
        l_ref[...] = jnp.zeros_like(l_ref)

    l_ref[...] += part


def _fuse_proj(e1, e2, wa, va, wf, w1a, w1b, bias):
    n, h = e1.shape
    return pl.pallas_call(
        _fuse_body,
        grid=(n // _BLK,),
        in_specs=[
            pl.BlockSpec((_BLK, h), lambda i: (i, 0)),
            pl.BlockSpec((_BLK, h), lambda i: (i, 0)),
            pl.BlockSpec((h, h), lambda i: (0, 0)),
            pl.BlockSpec((h, 1), lambda i: (0, 0)),
            pl.BlockSpec((h, h), lambda i: (0, 0)),
            pl.BlockSpec((h, h), lambda i: (0, 0)),
            pl.BlockSpec((h, h), lambda i: (0, 0)),
            pl.BlockSpec((1, h), lambda i: (0, 0)),
        ],
        out_specs=[
            pl.BlockSpec((_BLK, h), lambda i: (i, 0)),
            pl.BlockSpec((1, 1), lambda i: (0, 0)),
        ],
        out_shape=[
            jax.ShapeDtypeStruct((n, h), jnp.float32),
            jax.ShapeDtypeStruct((1, 1), jnp.float32),
        ],
        compiler_params=pltpu.CompilerParams(
            dimension_semantics=("arbitrary",)),
    )(e1, e2, wa, va, wf, w1a, w1b, bias)


def _pair_predict(pm, pd, mi, di, w2, b2v):
    mesh = plsc.VectorSubcoreMesh(core_axis_name="c", subcore_axis_name="s")
    cp = pltpu.CompilerParams()
    if "needs_layout_passes" in pltpu.CompilerParams.__dataclass_fields__:
        cp = dataclasses.replace(cp, needs_layout_passes=False)

    @pl.kernel(
        compiler_params=cp,
        out_type=jax.ShapeDtypeStruct((_T,), jnp.float32),
        mesh=mesh,
        scratch_types=[
            pltpu.VMEM((_PPW,), jnp.int32),
            pltpu.VMEM((_PPW,), jnp.int32),
            pltpu.VMEM((_CH, _H), jnp.float32),
            pltpu.VMEM((_CH, _H), jnp.float32),
            pltpu.VMEM((_CH, _H), jnp.float32),
            pltpu.VMEM((_CH, _H), jnp.float32),
            pltpu.VMEM((_PPW,), jnp.float32),
            pltpu.VMEM((_H,), jnp.float32),
            pltpu.VMEM((_L,), jnp.float32),
            pltpu.SemaphoreType.DMA,
            pltpu.SemaphoreType.DMA,
            pltpu.SemaphoreType.DMA,
            pltpu.SemaphoreType.DMA,
        ],
    )
    def body(pm_hbm, pd_hbm, mi_hbm, di_hbm, w2_hbm, b2_hbm, o_hbm,
             mi_v, di_v, pm0, pm1, pd0, pd1, out_v, w2_v, b2_v,
             sm0, sm1, sd0, sd1):
        wid = lax.axis_index("s") * 2 + lax.axis_index("c")
        base = wid * _PPW
        pltpu.sync_copy(mi_hbm.at[pl.ds(base, _PPW)], mi_v)
        pltpu.sync_copy(di_hbm.at[pl.ds(base, _PPW)], di_v)
        pltpu.sync_copy(w2_hbm, w2_v)
        pltpu.sync_copy(b2_hbm, b2_v)
        lane = lax.iota(jnp.int32, _L)
        b2vec = b2_v[...]
        bufs = ((pm0, pd0, sm0, sd0), (pm1, pd1, sm1, sd1))
        handles = [None, None]

        def start(c, b):
            sl = pl.ds(c * _CH, _CH)
            pm_b, pd_b, sem_m, sem_d = bufs[b]
            hm = pltpu.async_copy(pm_hbm.at[mi_v.at[sl]], pm_b, sem_m)
            hd = pltpu.async_copy(pd_hbm.at[di_v.at[sl]], pd_b, sem_d)
            handles[b] = (hm, hd)

        start(0, 0)
        for c in range(_NCH):
            b = c & 1
            pm_b, pd_b = bufs[b][0], bufs[b][1]
            handles[b][0].wait()
            handles[b][1].wait()
            if c + 1 < _NCH:
                start(c + 1, 1 - b)

            @pl.loop(0, _CH, step=_L)
            def _(g0, _c=c, _pm=pm_b, _pd=pd_b):
                def jblock(jb, accs):
                    sl = pl.ds(jb * _L, _L)
                    w2c = w2_v[sl]
                    new = []
                    for p in range(_L):
                        pmv = _pm[g0 + p, sl]
                        pdv = _pd[g0 + p, sl]
                        h = jnp.maximum(pmv + pdv, 0.0)
                        new.append(accs[p] + h * w2c)
                    return tuple(new)

                zero = jnp.zeros((_L,), jnp.float32)
                accs = lax.fori_loop(0, _H // _L, jblock, (zero,) * _L)
                sv = zero
                for p in range(_L):
                    sv = jnp.where(lane == p, jnp.sum(accs[p]), sv)
                logit = sv + b2vec
                out_v[pl.ds(_c * _CH + g0, _L)] = 1.0 / (1.0 + jnp.exp(-logit))

        pltpu.sync_copy(out_v, o_hbm.at[pl.ds(base, _PPW)])

    return body(pm, pd, mi, di, w2, b2v)


def kernel(mm_sim, dd_sim, xm, xd, train_data, Wm1, Wm2, Wd1, Wd2,
           Wa_m, va_m, Wa_d, va_d, Wf_m, Wf_d, W1, b1, W2, b2):
    m_idx = train_data[:, 0].astype(jnp.int32)
    d_idx = train_data[:, 1].astype(jnp.int32)

    em1, deg_m = _gcn(mm_sim, xm, Wm1)
    em2 = _gcn(mm_sim, em1, Wm2, deg=deg_m)
    ed1, deg_d = _gcn(dd_sim, xd, Wd1)
    ed2 = _gcn(dd_sim, ed1, Wd2, deg=deg_d)

    pm, lossc = _fuse_proj(em1, em2, Wa_m, va_m.reshape(_H, 1), Wf_m,
                           W1[0:_H], W1[_H:2 * _H], b1.reshape(1, _H))
    pd, lossd = _fuse_proj(ed1, ed2, Wa_d, va_d.reshape(_H, 1), Wf_d,
                           W1[2 * _H:3 * _H], W1[3 * _H:4 * _H],
                           jnp.zeros((1, _H), jnp.float32))

    pre = _pair_predict(pm, pd, m_idx, d_idx, W2[:, 0],
                        jnp.broadcast_to(b2, (_L,)))
    return (pre, lossc[0, 0], lossd[0, 0])


# speedup vs baseline: 1.2305x; 1.0473x over previous
import dataclasses
import functools

import jax
import jax.numpy as jnp
from jax import lax
from jax.experimental import pallas as pl
from jax.experimental.pallas import tpu as pltpu
from jax.experimental.pallas import tpu_sc as plsc

_N = 4096
_H = 512
_T = 16384
_BLK = 512

_NW = 32
_PPW = _T // _NW
_CH = 32
_NCH = _PPW // _CH
_L = 16


_GBLK = 512


def _gcn1_body(x_ref, w_ref, sim_ref, o_ref, deg_ref, xw_ref):
    @pl.when(pl.program_id(0) == 0)
    def _():
        xw_ref[...] = jnp.dot(x_ref[...], w_ref[...],
                              preferred_element_type=jnp.float32)

    s = sim_ref[...]
    acc = jnp.dot(s, xw_ref[...], preferred_element_type=jnp.float32)
    deg = jnp.sum(s, axis=1, keepdims=True) + 1e-8
    deg_ref[...] = deg
    o_ref[...] = jnp.maximum(acc / deg, 0.0)


def _gcn2_body(x_ref, w_ref, sim_ref, deg_ref, o_ref, xw_ref):
    @pl.when(pl.program_id(0) == 0)
    def _():
        xw_ref[...] = jnp.dot(x_ref[...], w_ref[...],
                              preferred_element_type=jnp.float32)

    acc = jnp.dot(sim_ref[...], xw_ref[...],
                  preferred_element_type=jnp.float32)
    o_ref[...] = jnp.maximum(acc / deg_ref[...], 0.0)


def _gcn(sim, x, w, deg=None):
    n, d = x.shape
    h = w.shape[1]
    full_x = pl.BlockSpec((n, d), lambda i: (0, 0))
    full_w = pl.BlockSpec((d, h), lambda i: (0, 0))
    sim_spec = pl.BlockSpec((_GBLK, n), lambda i: (i, 0))
    row_spec = pl.BlockSpec((_GBLK, h), lambda i: (i, 0))
    deg_spec = pl.BlockSpec((_GBLK, 1), lambda i: (i, 0))
    params = pltpu.CompilerParams(dimension_semantics=("arbitrary",))
    scratch = [pltpu.VMEM((n, h), jnp.float32)]
    if deg is None:
        return pl.pallas_call(
            _gcn1_body,
            grid=(n // _GBLK,),
            in_specs=[full_x, full_w, sim_spec],
            out_specs=[row_spec, deg_spec],
            out_shape=[
                jax.ShapeDtypeStruct((n, h), jnp.float32),
                jax.ShapeDtypeStruct((n, 1), jnp.float32),
            ],
            scratch_shapes=scratch,
            compiler_params=params,
        )(x, w, sim)
    return pl.pallas_call(
        _gcn2_body,
        grid=(n // _GBLK,),
        in_specs=[full_x, full_w, sim_spec, deg_spec],
        out_specs=row_spec,
        out_shape=jax.ShapeDtypeStruct((n, h), jnp.float32),
        scratch_shapes=scratch,
        compiler_params=params,
    )(x, w, sim, deg)


def _fuse_body(e1_ref, e2_ref, wa_ref, va_ref, wf_ref, w1a_ref, w1b_ref,
               bias_ref, p_ref, l_ref):
    e1 = e1_ref[...]
    e2 = e2_ref[...]
    wa = wa_ref[...]
    t1 = jnp.tanh(jnp.dot(e1, wa, preferred_element_type=jnp.float32))
    t2 = jnp.tanh(jnp.dot(e2, wa, preferred_element_type=jnp.float32))
    s1 = jnp.dot(t1, va_ref[...], preferred_element_type=jnp.float32)
    s2 = jnp.dot(t2, va_ref[...], preferred_element_type=jnp.float32)
    m = jnp.maximum(s1, s2)
    a1 = jnp.exp(s1 - m)
    a2 = jnp.exp(s2 - m)
    den = a1 + a2
    c1 = (a1 / den) * e1 + (a2 / den) * e2
    c2 = jnp.maximum(jnp.dot(c1, wf_ref[...],
                             preferred_element_type=jnp.float32), 0.0)
    p_ref[...] = (jnp.dot(c1, w1a_ref[...], preferred_element_type=jnp.float32)
                  + jnp.dot(c2, w1b_ref[...],
                            preferred_element_type=jnp.float32)
                  + bias_ref[...])

    q1 = jnp.sum(e1 * e1, axis=1, keepdims=True)
    q2 = jnp.sum(e2 * e2, axis=1, keepdims=True)
    dq = jnp.sum(e1 * e2, axis=1, keepdims=True)
    cos = dq / ((jnp.sqrt(q1) + 1e-8) * (jnp.sqrt(q2) + 1e-8))
    ls = jnp.minimum(cos, 0.0) - jnp.log(1.0 + jnp.exp(-jnp.abs(cos)))
    part = -jnp.sum(ls, axis=0, keepdims=True) / _N

    @pl.when(pl.program_id(0) == 0)
    def _():
        l_ref[...] = jnp.zeros_like(l_ref)

    l_ref[...] += part


def _fuse_proj(e1, e2, wa, va, wf, w1a, w1b, bias):
    n, h = e1.shape
    return pl.pallas_call(
        _fuse_body,
        grid=(n // _BLK,),
        in_specs=[
            pl.BlockSpec((_BLK, h), lambda i: (i, 0)),
            pl.BlockSpec((_BLK, h), lambda i: (i, 0)),
            pl.BlockSpec((h, h), lambda i: (0, 0)),
            pl.BlockSpec((h, 1), lambda i: (0, 0)),
            pl.BlockSpec((h, h), lambda i: (0, 0)),
            pl.BlockSpec((h, h), lambda i: (0, 0)),
            pl.BlockSpec((h, h), lambda i: (0, 0)),
            pl.BlockSpec((1, h), lambda i: (0, 0)),
        ],
        out_specs=[
            pl.BlockSpec((_BLK, h), lambda i: (i, 0)),
            pl.BlockSpec((1, 1), lambda i: (0, 0)),
        ],
        out_shape=[
            jax.ShapeDtypeStruct((n, h), jnp.float32),
            jax.ShapeDtypeStruct((1, 1), jnp.float32),
        ],
        compiler_params=pltpu.CompilerParams(
            dimension_semantics=("arbitrary",)),
    )(e1, e2, wa, va, wf, w1a, w1b, bias)


def _pair_predict(pm, pd, mi, di, w2, b2v):
    mesh = plsc.VectorSubcoreMesh(core_axis_name="c", subcore_axis_name="s")
    cp = pltpu.CompilerParams()
    if "needs_layout_passes" in pltpu.CompilerParams.__dataclass_fields__:
        cp = dataclasses.replace(cp, needs_layout_passes=False)

    @pl.kernel(
        compiler_params=cp,
        out_type=jax.ShapeDtypeStruct((_T,), jnp.float32),
        mesh=mesh,
        scratch_types=[
            pltpu.VMEM((_PPW,), jnp.int32),
            pltpu.VMEM((_PPW,), jnp.int32),
            pltpu.VMEM((_CH, _H), jnp.float32),
            pltpu.VMEM((_CH, _H), jnp.float32),
            pltpu.VMEM((_CH, _H), jnp.float32),
            pltpu.VMEM((_CH, _H), jnp.float32),
            pltpu.VMEM((_PPW,), jnp.float32),
            pltpu.VMEM((_H,), jnp.float32),
            pltpu.VMEM((_L,), jnp.float32),
            pltpu.SemaphoreType.DMA,
            pltpu.SemaphoreType.DMA,
            pltpu.SemaphoreType.DMA,
            pltpu.SemaphoreType.DMA,
        ],
    )
    def body(pm_hbm, pd_hbm, mi_hbm, di_hbm, w2_hbm, b2_hbm, o_hbm,
             mi_v, di_v, pm0, pm1, pd0, pd1, out_v, w2_v, b2_v,
             sm0, sm1, sd0, sd1):
        wid = lax.axis_index("s") * 2 + lax.axis_index("c")
        base = wid * _PPW
        pltpu.sync_copy(mi_hbm.at[pl.ds(base, _PPW)], mi_v)
        pltpu.sync_copy(di_hbm.at[pl.ds(base, _PPW)], di_v)
        pltpu.sync_copy(w2_hbm, w2_v)
        pltpu.sync_copy(b2_hbm, b2_v)
        lane = lax.iota(jnp.int32, _L)
        b2vec = b2_v[...]
        bufs = ((pm0, pd0, sm0, sd0), (pm1, pd1, sm1, sd1))
        handles = [None, None]

        def start(c, b):
            sl = pl.ds(c * _CH, _CH)
            pm_b, pd_b, sem_m, sem_d = bufs[b]
            hm = pltpu.async_copy(pm_hbm.at[mi_v.at[sl]], pm_b, sem_m)
            hd = pltpu.async_copy(pd_hbm.at[di_v.at[sl]], pd_b, sem_d)
            handles[b] = (hm, hd)

        start(0, 0)
        for c in range(_NCH):
            b = c & 1
            pm_b, pd_b = bufs[b][0], bufs[b][1]
            handles[b][0].wait()
            handles[b][1].wait()
            if c + 1 < _NCH:
                start(c + 1, 1 - b)

            @pl.loop(0, _CH, step=_L)
            def _(g0, _c=c, _pm=pm_b, _pd=pd_b):
                def jblock(jb, accs):
                    sl = pl.ds(jb * _L, _L)
                    w2c = w2_v[sl]
                    new = []
                    for p in range(_L):
                        pmv = _pm[g0 + p, sl]
                        pdv = _pd[g0 + p, sl]
                        h = jnp.maximum(pmv + pdv, 0.0)
                        new.append(accs[p] + h * w2c)
                    return tuple(new)

                zero = jnp.zeros((_L,), jnp.float32)
                accs = lax.fori_loop(0, _H // _L, jblock, (zero,) * _L)
                sv = zero
                for p in range(_L):
                    sv = jnp.where(lane == p, jnp.sum(accs[p]), sv)
                logit = sv + b2vec
                out_v[pl.ds(_c * _CH + g0, _L)] = 1.0 / (1.0 + jnp.exp(-logit))

        pltpu.sync_copy(out_v, o_hbm.at[pl.ds(base, _PPW)])

    return body(pm, pd, mi, di, w2, b2v)


def kernel(mm_sim, dd_sim, xm, xd, train_data, Wm1, Wm2, Wd1, Wd2,
           Wa_m, va_m, Wa_d, va_d, Wf_m, Wf_d, W1, b1, W2, b2):
    m_idx = train_data[:, 0].astype(jnp.int32)
    d_idx = train_data[:, 1].astype(jnp.int32)

    em1, deg_m = _gcn(mm_sim, xm, Wm1)
    em2 = _gcn(mm_sim, em1, Wm2, deg=deg_m)
    ed1, deg_d = _gcn(dd_sim, xd, Wd1)
    ed2 = _gcn(dd_sim, ed1, Wd2, deg=deg_d)

    pm, lossc = _fuse_proj(em1, em2, Wa_m, va_m.reshape(_H, 1), Wf_m,
                           W1[0:_H], W1[_H:2 * _H], b1.reshape(1, _H))
    pd, lossd = _fuse_proj(ed1, ed2, Wa_d, va_d.reshape(_H, 1), Wf_d,
                           W1[2 * _H:3 * _H], W1[3 * _H:4 * _H],
                           jnp.zeros((1, _H), jnp.float32))

    pre = _pair_predict(pm, pd, m_idx, d_idx, W2[:, 0],
                        jnp.broadcast_to(b2, (_L,)))
    return (pre, lossc[0, 0], lossd[0, 0])


# speedup vs baseline: 1.2917x; 1.0497x over previous
import dataclasses
import functools

import jax
import jax.numpy as jnp
from jax import lax
from jax.experimental import pallas as pl
from jax.experimental.pallas import tpu as pltpu
from jax.experimental.pallas import tpu_sc as plsc

_N = 4096
_H = 512
_T = 16384
_BLK = 512

_NW = 32
_PPW = _T // _NW
_CH = 32
_NCH = _PPW // _CH
_L = 16


_GBLK = 512


def _gcn1_body(x_ref, w_ref, sim_ref, o_ref, deg_ref, xw_ref):
    @pl.when(pl.program_id(0) == 0)
    def _():
        xw_ref[...] = jnp.dot(x_ref[...], w_ref[...],
                              preferred_element_type=jnp.float32)

    s = sim_ref[...]
    acc = jnp.dot(s, xw_ref[...], preferred_element_type=jnp.float32)
    deg = jnp.sum(s, axis=1, keepdims=True) + 1e-8
    deg_ref[...] = deg
    o_ref[...] = jnp.maximum(acc / deg, 0.0)


def _gcn2_body(x_ref, w_ref, sim_ref, deg_ref, o_ref, xw_ref):
    @pl.when(pl.program_id(0) == 0)
    def _():
        xw_ref[...] = jnp.dot(x_ref[...], w_ref[...],
                              preferred_element_type=jnp.float32)

    acc = jnp.dot(sim_ref[...], xw_ref[...],
                  preferred_element_type=jnp.float32)
    o_ref[...] = jnp.maximum(acc / deg_ref[...], 0.0)


def _gcn(sim, x, w, deg=None):
    n, d = x.shape
    h = w.shape[1]
    full_x = pl.BlockSpec((n, d), lambda i: (0, 0))
    full_w = pl.BlockSpec((d, h), lambda i: (0, 0))
    sim_spec = pl.BlockSpec((_GBLK, n), lambda i: (i, 0))
    row_spec = pl.BlockSpec((_GBLK, h), lambda i: (i, 0))
    deg_spec = pl.BlockSpec((_GBLK, 1), lambda i: (i, 0))
    params = pltpu.CompilerParams(dimension_semantics=("arbitrary",))
    scratch = [pltpu.VMEM((n, h), jnp.float32)]
    if deg is None:
        return pl.pallas_call(
            _gcn1_body,
            grid=(n // _GBLK,),
            in_specs=[full_x, full_w, sim_spec],
            out_specs=[row_spec, deg_spec],
            out_shape=[
                jax.ShapeDtypeStruct((n, h), jnp.float32),
                jax.ShapeDtypeStruct((n, 1), jnp.float32),
            ],
            scratch_shapes=scratch,
            compiler_params=params,
        )(x, w, sim)
    return pl.pallas_call(
        _gcn2_body,
        grid=(n // _GBLK,),
        in_specs=[full_x, full_w, sim_spec, deg_spec],
        out_specs=row_spec,
        out_shape=jax.ShapeDtypeStruct((n, h), jnp.float32),
        scratch_shapes=scratch,
        compiler_params=params,
    )(x, w, sim, deg)


def _fuse_body(e1_ref, e2_ref, wa_ref, va_ref, wf_ref, w1a_ref, w1b_ref,
               bias_ref, p_ref, l_ref):
    e1 = e1_ref[...]
    e2 = e2_ref[...]
    wa = wa_ref[...]
    t1 = jnp.tanh(jnp.dot(e1, wa, preferred_element_type=jnp.float32))
    t2 = jnp.tanh(jnp.dot(e2, wa, preferred_element_type=jnp.float32))
    s1 = jnp.dot(t1, va_ref[...], preferred_element_type=jnp.float32)
    s2 = jnp.dot(t2, va_ref[...], preferred_element_type=jnp.float32)
    m = jnp.maximum(s1, s2)
    a1 = jnp.exp(s1 - m)
    a2 = jnp.exp(s2 - m)
    den = a1 + a2
    c1 = (a1 / den) * e1 + (a2 / den) * e2
    c2 = jnp.maximum(jnp.dot(c1, wf_ref[...],
                             preferred_element_type=jnp.float32), 0.0)
    pb = (jnp.dot(c1, w1a_ref[...], preferred_element_type=jnp.float32)
          + jnp.dot(c2, w1b_ref[...], preferred_element_type=jnp.float32)
          + bias_ref[...]).astype(jnp.bfloat16)
    hh = pb.shape[1] // 2
    lo = jax.lax.bitcast_convert_type(pb[:, :hh],
                                      jnp.uint16).astype(jnp.uint32)
    hi = jax.lax.bitcast_convert_type(pb[:, hh:],
                                      jnp.uint16).astype(jnp.uint32)
    p_ref[...] = jax.lax.bitcast_convert_type(lo | (hi << 16), jnp.int32)

    q1 = jnp.sum(e1 * e1, axis=1, keepdims=True)
    q2 = jnp.sum(e2 * e2, axis=1, keepdims=True)
    dq = jnp.sum(e1 * e2, axis=1, keepdims=True)
    cos = dq / ((jnp.sqrt(q1) + 1e-8) * (jnp.sqrt(q2) + 1e-8))
    ls = jnp.minimum(cos, 0.0) - jnp.log(1.0 + jnp.exp(-jnp.abs(cos)))
    part = -jnp.sum(ls, axis=0, keepdims=True) / _N

    @pl.when(pl.program_id(0) == 0)
    def _():
        l_ref[...] = jnp.zeros_like(l_ref)

    l_ref[...] += part


def _fuse_proj(e1, e2, wa, va, wf, w1a, w1b, bias):
    n, h = e1.shape
    return pl.pallas_call(
        _fuse_body,
        grid=(n // _BLK,),
        in_specs=[
            pl.BlockSpec((_BLK, h), lambda i: (i, 0)),
            pl.BlockSpec((_BLK, h), lambda i: (i, 0)),
            pl.BlockSpec((h, h), lambda i: (0, 0)),
            pl.BlockSpec((h, 1), lambda i: (0, 0)),
            pl.BlockSpec((h, h), lambda i: (0, 0)),
            pl.BlockSpec((h, h), lambda i: (0, 0)),
            pl.BlockSpec((h, h), lambda i: (0, 0)),
            pl.BlockSpec((1, h), lambda i: (0, 0)),
        ],
        out_specs=[
            pl.BlockSpec((_BLK, h // 2), lambda i: (i, 0)),
            pl.BlockSpec((1, 1), lambda i: (0, 0)),
        ],
        out_shape=[
            jax.ShapeDtypeStruct((n, h // 2), jnp.int32),
            jax.ShapeDtypeStruct((1, 1), jnp.float32),
        ],
        compiler_params=pltpu.CompilerParams(
            dimension_semantics=("arbitrary",)),
    )(e1, e2, wa, va, wf, w1a, w1b, bias)


def _pair_predict(pm, pd, mi, di, w2, b2v):
    mesh = plsc.VectorSubcoreMesh(core_axis_name="c", subcore_axis_name="s")
    cp = pltpu.CompilerParams()
    if "needs_layout_passes" in pltpu.CompilerParams.__dataclass_fields__:
        cp = dataclasses.replace(cp, needs_layout_passes=False)

    @pl.kernel(
        compiler_params=cp,
        out_type=jax.ShapeDtypeStruct((_T,), jnp.float32),
        mesh=mesh,
        scratch_types=[
            pltpu.VMEM((_PPW,), jnp.int32),
            pltpu.VMEM((_PPW,), jnp.int32),
            pltpu.VMEM((_CH, _H // 2), jnp.int32),
            pltpu.VMEM((_CH, _H // 2), jnp.int32),
            pltpu.VMEM((_CH, _H // 2), jnp.int32),
            pltpu.VMEM((_CH, _H // 2), jnp.int32),
            pltpu.VMEM((_PPW,), jnp.float32),
            pltpu.VMEM((_H,), jnp.float32),
            pltpu.VMEM((_L,), jnp.float32),
            pltpu.SemaphoreType.DMA,
            pltpu.SemaphoreType.DMA,
            pltpu.SemaphoreType.DMA,
            pltpu.SemaphoreType.DMA,
        ],
    )
    def body(pm_hbm, pd_hbm, mi_hbm, di_hbm, w2_hbm, b2_hbm, o_hbm,
             mi_v, di_v, pm0, pm1, pd0, pd1, out_v, w2_v, b2_v,
             sm0, sm1, sd0, sd1):
        wid = lax.axis_index("s") * 2 + lax.axis_index("c")
        base = wid * _PPW
        pltpu.sync_copy(mi_hbm.at[pl.ds(base, _PPW)], mi_v)
        pltpu.sync_copy(di_hbm.at[pl.ds(base, _PPW)], di_v)
        pltpu.sync_copy(w2_hbm, w2_v)
        pltpu.sync_copy(b2_hbm, b2_v)
        lane = lax.iota(jnp.int32, _L)
        b2vec = b2_v[...]
        bufs = ((pm0, pd0, sm0, sd0), (pm1, pd1, sm1, sd1))
        handles = [None, None]

        def start(c, b):
            sl = pl.ds(c * _CH, _CH)
            pm_b, pd_b, sem_m, sem_d = bufs[b]
            hm = pltpu.async_copy(pm_hbm.at[mi_v.at[sl]], pm_b, sem_m)
            hd = pltpu.async_copy(pd_hbm.at[di_v.at[sl]], pd_b, sem_d)
            handles[b] = (hm, hd)

        start(0, 0)
        for c in range(_NCH):
            b = c & 1
            pm_b, pd_b = bufs[b][0], bufs[b][1]
            handles[b][0].wait()
            handles[b][1].wait()
            if c + 1 < _NCH:
                start(c + 1, 1 - b)

            @pl.loop(0, _CH, step=_L)
            def _(g0, _c=c, _pm=pm_b, _pd=pd_b):
                def jblock(jb, accs):
                    sl = pl.ds(jb * _L, _L)
                    w2e = w2_v[pl.ds(jb * _L, _L)]
                    w2o = w2_v[pl.ds(_H // 2 + jb * _L, _L)]
                    zb = jnp.zeros((2 * _L,), jnp.bfloat16)
                    new = []
                    for p in range(_L):
                        pmv = plsc.bitcast(_pm[g0 + p, sl], jnp.bfloat16)
                        pdv = plsc.bitcast(_pd[g0 + p, sl], jnp.bfloat16)
                        h = jnp.maximum(pmv + pdv, zb)
                        he, ho = plsc.unpack(
                            h, format=plsc.PackFormat.INTERLEAVED)
                        new.append(accs[p] + (he * w2e + ho * w2o))
                    return tuple(new)

                zero = jnp.zeros((_L,), jnp.float32)
                accs = lax.fori_loop(0, _H // (2 * _L), jblock, (zero,) * _L)
                sv = zero
                for p in range(_L):
                    sv = jnp.where(lane == p, jnp.sum(accs[p]), sv)
                logit = sv + b2vec
                out_v[pl.ds(_c * _CH + g0, _L)] = 1.0 / (1.0 + jnp.exp(-logit))

        pltpu.sync_copy(out_v, o_hbm.at[pl.ds(base, _PPW)])

    return body(pm, pd, mi, di, w2, b2v)


def kernel(mm_sim, dd_sim, xm, xd, train_data, Wm1, Wm2, Wd1, Wd2,
           Wa_m, va_m, Wa_d, va_d, Wf_m, Wf_d, W1, b1, W2, b2):
    m_idx = train_data[:, 0].astype(jnp.int32)
    d_idx = train_data[:, 1].astype(jnp.int32)

    em1, deg_m = _gcn(mm_sim, xm, Wm1)
    em2 = _gcn(mm_sim, em1, Wm2, deg=deg_m)
    ed1, deg_d = _gcn(dd_sim, xd, Wd1)
    ed2 = _gcn(dd_sim, ed1, Wd2, deg=deg_d)

    pm, lossc = _fuse_proj(em1, em2, Wa_m, va_m.reshape(_H, 1), Wf_m,
                           W1[0:_H], W1[_H:2 * _H], b1.reshape(1, _H))
    pd, lossd = _fuse_proj(ed1, ed2, Wa_d, va_d.reshape(_H, 1), Wf_d,
                           W1[2 * _H:3 * _H], W1[3 * _H:4 * _H],
                           jnp.zeros((1, _H), jnp.float32))

    pre = _pair_predict(pm, pd, m_idx, d_idx, W2[:, 0],
                        jnp.broadcast_to(b2, (_L,)))
    return (pre, lossc[0, 0], lossd[0, 0])
